# Initial kernel scaffold; baseline (speedup 1.0000x reference)
import jax
import jax.numpy as jnp
from jax.experimental import pallas as pl


def kernel(x, edge_index, batch, Wrel0, brel0, Wroot0, Wrel1, brel1, Wroot1, Wrel2, brel2, Wroot2, Wlin1, blin1, Wlin2, blin2):
    raise NotImplementedError("write your pallas kernel here")


# speedup vs baseline: 4.6271x; 4.6271x over previous
import functools

import jax
import jax.numpy as jnp
from jax import lax
from jax.experimental import pallas as pl
from jax.experimental.pallas import tpu as pltpu
from jax.experimental.pallas import tpu_sc as plsc

N = 10000
E = 320000
D = 128
G = 64
OUT = 10

NC = 2
NS = 16
NW = NC * NS
CHUNK = 80
EDGES_PER_TILE = E // NW
NCHUNK = EDGES_PER_TILE // CHUNK
NPAD = 10240
ROWS_PER_TILE = NPAD // NS

BR = 1000
NBLK = N // BR


@functools.partial(
    pl.kernel,
    out_type=jax.ShapeDtypeStruct((NC, NPAD, D), jnp.float32),
    mesh=plsc.VectorSubcoreMesh(core_axis_name="c", subcore_axis_name="s"),
    scratch_types=[
        pltpu.VMEM((CHUNK,), jnp.int32),
        pltpu.VMEM((CHUNK,), jnp.int32),
        pltpu.VMEM((CHUNK, D), jnp.float32),
        pltpu.VMEM_SHARED((NPAD, D), jnp.float32),
        pltpu.SemaphoreType.DMA,
    ],
)
def _segsum_sc(x_hbm, src_hbm, dst_hbm, zeros_hbm, out_hbm,
               src_v, dst_v, rows_v, acc_sh, sem):
    c = lax.axis_index("c")
    s = lax.axis_index("s")
    wid = c * NS + s

    pltpu.sync_copy(zeros_hbm.at[pl.ds(s * ROWS_PER_TILE, ROWS_PER_TILE)],
                    acc_sh.at[pl.ds(s * ROWS_PER_TILE, ROWS_PER_TILE)])
    plsc.subcore_barrier()

    def body(i, carry):
        base = wid * EDGES_PER_TILE + i * CHUNK
        pltpu.sync_copy(src_hbm.at[pl.ds(base, CHUNK)], src_v)
        pltpu.sync_copy(dst_hbm.at[pl.ds(base, CHUNK)], dst_v)
        pltpu.async_copy(x_hbm.at[src_v], rows_v, sem).wait()
        pltpu.sync_copy(rows_v, acc_sh.at[dst_v], add=True)
        return carry

    lax.fori_loop(0, NCHUNK, body, 0)
    plsc.subcore_barrier()

    pltpu.sync_copy(acc_sh.at[pl.ds(s * ROWS_PER_TILE, ROWS_PER_TILE)],
                    out_hbm.at[c, pl.ds(s * ROWS_PER_TILE, ROWS_PER_TILE)])


def _layer_body(parts_ref, x_ref, wrel_ref, brel_ref, wroot_ref, o_ref):
    agg = parts_ref[0] + parts_ref[1]
    acc = jnp.dot(agg, wrel_ref[...], preferred_element_type=jnp.float32)
    acc += jnp.dot(x_ref[...], wroot_ref[...], preferred_element_type=jnp.float32)
    acc += brel_ref[...]
    o_ref[...] = jnp.maximum(acc, 0.0)


def _layer_tc(parts, x, wrel, brel, wroot):
    return pl.pallas_call(
        _layer_body,
        grid=(NBLK,),
        in_specs=[
            pl.BlockSpec((NC, BR, D), lambda i: (0, i, 0)),
            pl.BlockSpec((BR, D), lambda i: (i, 0)),
            pl.BlockSpec((D, D), lambda i: (0, 0)),
            pl.BlockSpec((1, D), lambda i: (0, 0)),
            pl.BlockSpec((D, D), lambda i: (0, 0)),
        ],
        out_specs=pl.BlockSpec((BR, D), lambda i: (i, 0)),
        out_shape=jax.ShapeDtypeStruct((N, D), jnp.float32),
    )(parts, x, wrel, brel.reshape(1, D), wroot)


def _final_body(parts_ref, x_ref, wrel_ref, brel_ref, wroot_ref,
                wlin1_ref, blin1_ref, wlin2_ref, blin2_ref, batch_ref, o_ref):
    agg = parts_ref[0] + parts_ref[1]
    h = jnp.dot(agg, wrel_ref[...], preferred_element_type=jnp.float32)
    h += jnp.dot(x_ref[...], wroot_ref[...], preferred_element_type=jnp.float32)
    h += brel_ref[...]
    h = jnp.maximum(h, 0.0)
    h = jnp.maximum(
        jnp.dot(h, wlin1_ref[...], preferred_element_type=jnp.float32)
        + blin1_ref[...], 0.0)
    y = jnp.dot(h, wlin2_ref[...], preferred_element_type=jnp.float32)
    y += blin2_ref[...]
    seg = lax.broadcasted_iota(jnp.int32, (BR, G), 1)
    mask = (batch_ref[...] == seg).astype(jnp.float32)
    contrib = lax.dot_general(mask, y, (((0,), (0,)), ((), ())),
                              preferred_element_type=jnp.float32)

    @pl.when(pl.program_id(0) == 0)
    def _():
        o_ref[...] = jnp.zeros_like(o_ref)

    o_ref[...] += contrib


def _final_tc(parts, x, wrel, brel, wroot, wlin1, blin1, wlin2, blin2, batch):
    return pl.pallas_call(
        _final_body,
        grid=(NBLK,),
        in_specs=[
            pl.BlockSpec((NC, BR, D), lambda i: (0, i, 0)),
            pl.BlockSpec((BR, D), lambda i: (i, 0)),
            pl.BlockSpec((D, D), lambda i: (0, 0)),
            pl.BlockSpec((1, D), lambda i: (0, 0)),
            pl.BlockSpec((D, D), lambda i: (0, 0)),
            pl.BlockSpec((D, D), lambda i: (0, 0)),
            pl.BlockSpec((1, D), lambda i: (0, 0)),
            pl.BlockSpec((D, OUT), lambda i: (0, 0)),
            pl.BlockSpec((1, OUT), lambda i: (0, 0)),
            pl.BlockSpec((BR, 1), lambda i: (i, 0)),
        ],
        out_specs=pl.BlockSpec((G, OUT), lambda i: (0, 0)),
        out_shape=jax.ShapeDtypeStruct((G, OUT), jnp.float32),
    )(parts, x, wrel, brel.reshape(1, D), wroot,
      wlin1, blin1.reshape(1, D), wlin2, blin2.reshape(1, OUT),
      batch.reshape(N, 1))


def kernel(x, edge_index, batch,
           Wrel0, brel0, Wroot0,
           Wrel1, brel1, Wroot1,
           Wrel2, brel2, Wroot2,
           Wlin1, blin1, Wlin2, blin2):
    src = edge_index[0]
    dst = edge_index[1]
    zeros = jnp.zeros((NPAD, D), jnp.float32)

    parts = _segsum_sc(x, src, dst, zeros)
    h = _layer_tc(parts, x, Wrel0, brel0, Wroot0)
    parts = _segsum_sc(h, src, dst, zeros)
    h = _layer_tc(parts, h, Wrel1, brel1, Wroot1)
    parts = _segsum_sc(h, src, dst, zeros)
    return _final_tc(parts, h, Wrel2, brel2, Wroot2,
                     Wlin1, blin1, Wlin2, blin2, batch)
